# baseline (speedup 1.0000x reference)
import functools

import jax
import jax.numpy as jnp
from jax import lax
from jax.experimental import pallas as pl
from jax.experimental.pallas import tpu as pltpu
from jax.experimental.pallas import tpu_sc as plsc

N = 10000
E = 320000
D = 128
G = 64
NCLS = 10
NC, NS = 2, 16
NW = NC * NS
NPAD = 10240
RPS = NPAD // NS
RPT = NPAD // NW
TCH = 80
C = 80
NCHUNK = 128
EPAD = NW * NCHUNK * C
EPT = NCHUNK * C
EHALF = EPT // 2
NH = NCHUNK // 2

_mesh = plsc.VectorSubcoreMesh(core_axis_name="c", subcore_axis_name="s")


@functools.partial(
    pl.kernel,
    out_type=(
        jax.ShapeDtypeStruct((NPAD, D), jnp.float32),
        jax.ShapeDtypeStruct((NC, NPAD, D), jnp.float32),
    ),
    mesh=_mesh,
    scratch_types=[
        pltpu.VMEM((EHALF,), jnp.int32),
        pltpu.VMEM((TCH,), jnp.int32),
        pltpu.VMEM((TCH, D), jnp.float32),
        pltpu.VMEM((C, D), jnp.float32),
        pltpu.VMEM_SHARED((NPAD, D), jnp.float32),
        pltpu.SemaphoreType.DMA,
        pltpu.SemaphoreType.DMA,
    ],
)
def _embed_deg_k(tok_hbm, emb_hbm, dst_hbm, z_hbm, ones_hbm,
                 x_hbm, deg_hbm, didx_v, tidx_v, trows_v, ones_v, deg_sh,
                 sem0, sem1):
    cid = lax.axis_index("c")
    sid = lax.axis_index("s")
    wid = sid * NC + cid
    zoff = pl.multiple_of(sid * RPS, 8)
    pltpu.sync_copy(z_hbm.at[pl.ds(zoff, RPS)], deg_sh.at[pl.ds(zoff, RPS)])
    pltpu.sync_copy(ones_hbm, ones_v)
    plsc.subcore_barrier()

    K = 4

    def deg_half(p):
        ebase = pl.multiple_of(wid * EPT + p * EHALF, 8)
        pltpu.sync_copy(dst_hbm.at[pl.ds(ebase, EHALF)], didx_v)

        def edge_body(j, _):
            off = pl.multiple_of(j * C, 8)
            pltpu.async_copy(ones_v, deg_sh.at[didx_v.at[pl.ds(off, C)]],
                             sem0, add=True)

            @pl.when(j >= K)
            def _wait():
                pltpu.make_async_copy(
                    ones_v, deg_sh.at[didx_v.at[pl.ds(off, C)]],
                    sem0).wait()

            return 0

        lax.fori_loop(0, NCHUNK // 2, edge_body, 0)

        def drain_body(j, _):
            pltpu.make_async_copy(
                ones_v, deg_sh.at[didx_v.at[pl.ds(0, C)]], sem0).wait()
            return 0

        lax.fori_loop(0, K, drain_body, 0)

    deg_half(0)
    deg_half(1)

    rbase = wid * RPT

    def tok_body(i, _):
        off = pl.multiple_of(rbase + i * TCH, 8)
        pltpu.sync_copy(tok_hbm.at[pl.ds(off, TCH)], tidx_v)
        pltpu.async_copy(emb_hbm.at[tidx_v], trows_v, sem1).wait()
        pltpu.sync_copy(trows_v, x_hbm.at[pl.ds(off, TCH)])
        return 0

    lax.fori_loop(0, RPT // TCH, tok_body, 0)

    plsc.subcore_barrier()
    pltpu.sync_copy(deg_sh.at[pl.ds(zoff, RPS)],
                    deg_hbm.at[cid, pl.ds(zoff, RPS)])


@functools.partial(
    pl.kernel,
    out_type=jax.ShapeDtypeStruct((NC, NPAD, D), jnp.float32),
    mesh=_mesh,
    scratch_types=[
        pltpu.VMEM((C,), jnp.int32),
        pltpu.VMEM((C,), jnp.int32),
        pltpu.VMEM((C,), jnp.int32),
        pltpu.VMEM((C, D), jnp.float32),
        pltpu.VMEM((C, D), jnp.float32),
        pltpu.VMEM_SHARED((NPAD, D), jnp.float32),
        pltpu.SemaphoreType.DMA,
        pltpu.SemaphoreType.DMA,
    ],
)
def _agg_k(src_hbm, dst_hbm, h_hbm, z_hbm,
           agg_hbm, s0, d0, d1, buf0, buf1, agg_sh, gsem, ssem):
    cid = lax.axis_index("c")
    sid = lax.axis_index("s")
    wid = sid * NC + cid
    zoff = pl.multiple_of(sid * RPS, 8)
    pltpu.sync_copy(z_hbm.at[pl.ds(zoff, RPS)], agg_sh.at[pl.ds(zoff, RPS)])
    plsc.subcore_barrier()

    ebase = wid * EPT

    def body(i, _):
        off = pl.multiple_of(ebase + i * C, 8)
        pltpu.sync_copy(src_hbm.at[pl.ds(off, C)], s0)
        pltpu.sync_copy(dst_hbm.at[pl.ds(off, C)], d0)
        pltpu.async_copy(h_hbm.at[s0], buf0, gsem).wait()
        pltpu.sync_copy(buf0, agg_sh.at[d0], add=True)
        return 0

    lax.fori_loop(0, NCHUNK, body, 0)

    plsc.subcore_barrier()
    pltpu.sync_copy(agg_sh.at[pl.ds(zoff, RPS)],
                    agg_hbm.at[cid, pl.ds(zoff, RPS)])


_BLK = 256


def _sage_body(h_ref, a0_ref, a1_ref, d0_ref, d1_ref, ws_ref, wn_ref, b_ref,
               o_ref):
    deg = jnp.maximum(d0_ref[:, 0:1] + d1_ref[:, 0:1], 1.0)
    agg = (a0_ref[...] + a1_ref[...]) / deg
    acc = jnp.dot(h_ref[...], ws_ref[...], preferred_element_type=jnp.float32)
    acc += jnp.dot(agg, wn_ref[...], preferred_element_type=jnp.float32)
    o_ref[...] = jnp.maximum(acc + b_ref[...], 0.0)


def _sage(h, a0, a1, d0, d1, Ws, Wn, b):
    grid = (NPAD // _BLK,)
    return pl.pallas_call(
        _sage_body,
        grid=grid,
        in_specs=[
            pl.BlockSpec((_BLK, D), lambda i: (i, 0)),
            pl.BlockSpec((_BLK, D), lambda i: (i, 0)),
            pl.BlockSpec((_BLK, D), lambda i: (i, 0)),
            pl.BlockSpec((_BLK, D), lambda i: (i, 0)),
            pl.BlockSpec((_BLK, D), lambda i: (i, 0)),
            pl.BlockSpec((D, D), lambda i: (0, 0)),
            pl.BlockSpec((D, D), lambda i: (0, 0)),
            pl.BlockSpec((1, D), lambda i: (0, 0)),
        ],
        out_specs=pl.BlockSpec((_BLK, D), lambda i: (i, 0)),
        out_shape=jax.ShapeDtypeStruct((NPAD, D), jnp.float32),
    )(h, a0, a1, d0, d1, Ws, Wn, b)


def _cls_body(h_ref, gid_ref, wc1_ref, bc1_ref, wc2_ref, bc2_ref,
              o_ref, pooled_ref):
    def body(g, _):
        mask = gid_ref[...] == g
        vals = jnp.where(mask, h_ref[...], -jnp.inf)
        pooled_ref[pl.ds(g, 1), :] = jnp.max(vals, axis=0, keepdims=True)
        return 0

    lax.fori_loop(0, G, body, 0)
    pooled = pooled_ref[...]
    pooled = jnp.where(jnp.isfinite(pooled), pooled, 0.0)
    hid = jnp.dot(pooled, wc1_ref[...], preferred_element_type=jnp.float32)
    hid = jnp.maximum(hid + bc1_ref[...], 0.0)
    o_ref[...] = jnp.dot(hid, wc2_ref[...],
                         preferred_element_type=jnp.float32) + bc2_ref[...]


def _classifier(h2, gid, Wc1, bc1, Wc2p, bc2p):
    return pl.pallas_call(
        _cls_body,
        out_shape=jax.ShapeDtypeStruct((G, 128), jnp.float32),
        scratch_shapes=[pltpu.VMEM((G, D), jnp.float32)],
    )(h2, gid, Wc1, bc1, Wc2p, bc2p)


@jax.jit
def kernel(token_ids, edge_index, graph_ids, emb, W_self1, W_neigh1, b1,
           W_self2, W_neigh2, b2, Wc1, bc1, Wc2, bc2):
    tok = jnp.concatenate(
        [token_ids, jnp.zeros((NPAD - N,), jnp.int32)]).astype(jnp.int32)
    src = jnp.concatenate(
        [edge_index[0].astype(jnp.int32),
         jnp.zeros((EPAD - E,), jnp.int32)])
    dst = jnp.concatenate(
        [edge_index[1].astype(jnp.int32),
         N + jnp.arange(EPAD - E, dtype=jnp.int32) % (NPAD - N)])
    gid = jnp.concatenate(
        [graph_ids, jnp.full((NPAD - N,), G, jnp.int32)]).reshape(NPAD, 1)
    z = jnp.zeros((NPAD, D), jnp.float32)
    ones = jnp.ones((C, D), jnp.float32)

    x, deg = _embed_deg_k(tok, emb, dst, z, ones)
    agg1 = _agg_k(src, dst, x, z)
    h1 = _sage(x, agg1[0], agg1[1], deg[0], deg[1],
               W_self1, W_neigh1, b1.reshape(1, D))
    agg2 = _agg_k(src, dst, h1, z)
    h2 = _sage(h1, agg2[0], agg2[1], deg[0], deg[1],
               W_self2, W_neigh2, b2.reshape(1, D))

    Wc2p = jnp.pad(Wc2, ((0, 0), (0, 128 - NCLS)))
    bc2p = jnp.pad(bc2, (0, 128 - NCLS)).reshape(1, 128)
    logits = _classifier(h2, gid, Wc1, bc1.reshape(1, D), Wc2p, bc2p)
    return logits[:, :NCLS]


# speedup vs baseline: 1.7420x; 1.7420x over previous
import functools

import jax
import jax.numpy as jnp
from jax import lax
from jax.experimental import pallas as pl
from jax.experimental.pallas import tpu as pltpu
from jax.experimental.pallas import tpu_sc as plsc

N = 10000
E = 320000
D = 128
G = 64
NCLS = 10
NC, NS = 2, 16
NW = NC * NS
NPAD = 10240
RPS = NPAD // NS
RPT = NPAD // NW
TCH = 80
C = 80
NCHUNK = 128
EPAD = NW * NCHUNK * C
EPT = NCHUNK * C
EHALF = EPT // 2
NH = NCHUNK // 2

_mesh = plsc.VectorSubcoreMesh(core_axis_name="c", subcore_axis_name="s")


@functools.partial(
    pl.kernel,
    out_type=(
        jax.ShapeDtypeStruct((NPAD, D), jnp.float32),
        jax.ShapeDtypeStruct((NC, NPAD, D), jnp.float32),
    ),
    mesh=_mesh,
    scratch_types=[
        pltpu.VMEM((EHALF,), jnp.int32),
        pltpu.VMEM((TCH,), jnp.int32),
        pltpu.VMEM((TCH, D), jnp.float32),
        pltpu.VMEM((C, D), jnp.float32),
        pltpu.VMEM_SHARED((NPAD, D), jnp.float32),
        pltpu.SemaphoreType.DMA,
        pltpu.SemaphoreType.DMA,
    ],
)
def _embed_deg_k(tok_hbm, emb_hbm, dst_hbm, z_hbm, ones_hbm,
                 x_hbm, deg_hbm, didx_v, tidx_v, trows_v, ones_v, deg_sh,
                 sem0, sem1):
    cid = lax.axis_index("c")
    sid = lax.axis_index("s")
    wid = sid * NC + cid
    zoff = pl.multiple_of(sid * RPS, 8)
    pltpu.sync_copy(z_hbm.at[pl.ds(zoff, RPS)], deg_sh.at[pl.ds(zoff, RPS)])
    pltpu.sync_copy(ones_hbm, ones_v)
    plsc.subcore_barrier()

    K = 4

    def deg_half(p):
        ebase = pl.multiple_of(wid * EPT + p * EHALF, 8)
        pltpu.sync_copy(dst_hbm.at[pl.ds(ebase, EHALF)], didx_v)

        def edge_body(j, _):
            off = pl.multiple_of(j * C, 8)
            pltpu.async_copy(ones_v, deg_sh.at[didx_v.at[pl.ds(off, C)]],
                             sem0, add=True)

            @pl.when(j >= K)
            def _wait():
                pltpu.make_async_copy(
                    ones_v, deg_sh.at[didx_v.at[pl.ds(off, C)]],
                    sem0).wait()

            return 0

        lax.fori_loop(0, NCHUNK // 2, edge_body, 0)

        def drain_body(j, _):
            pltpu.make_async_copy(
                ones_v, deg_sh.at[didx_v.at[pl.ds(0, C)]], sem0).wait()
            return 0

        lax.fori_loop(0, K, drain_body, 0)

    deg_half(0)
    deg_half(1)

    rbase = wid * RPT

    def tok_body(i, _):
        off = pl.multiple_of(rbase + i * TCH, 8)
        pltpu.sync_copy(tok_hbm.at[pl.ds(off, TCH)], tidx_v)
        pltpu.async_copy(emb_hbm.at[tidx_v], trows_v, sem1).wait()
        pltpu.sync_copy(trows_v, x_hbm.at[pl.ds(off, TCH)])
        return 0

    lax.fori_loop(0, RPT // TCH, tok_body, 0)

    plsc.subcore_barrier()
    pltpu.sync_copy(deg_sh.at[pl.ds(zoff, RPS)],
                    deg_hbm.at[cid, pl.ds(zoff, RPS)])


@functools.partial(
    pl.kernel,
    out_type=jax.ShapeDtypeStruct((NC, NPAD, D), jnp.float32),
    mesh=_mesh,
    scratch_types=[
        pltpu.VMEM((C,), jnp.int32),
        pltpu.VMEM((C,), jnp.int32),
        pltpu.VMEM((C,), jnp.int32),
        pltpu.VMEM((C, D), jnp.float32),
        pltpu.VMEM((C, D), jnp.float32),
        pltpu.VMEM_SHARED((NPAD, D), jnp.float32),
        pltpu.SemaphoreType.DMA,
        pltpu.SemaphoreType.DMA,
    ],
)
def _agg_k(src_hbm, dst_hbm, h_hbm, z_hbm,
           agg_hbm, s0, d0, d1, buf0, buf1, agg_sh, gsem, ssem):
    cid = lax.axis_index("c")
    sid = lax.axis_index("s")
    wid = sid * NC + cid
    zoff = pl.multiple_of(sid * RPS, 8)
    pltpu.sync_copy(z_hbm.at[pl.ds(zoff, RPS)], agg_sh.at[pl.ds(zoff, RPS)])
    plsc.subcore_barrier()

    ebase = wid * EPT

    def body(i, _):
        off = pl.multiple_of(ebase + i * C, 8)
        pltpu.sync_copy(src_hbm.at[pl.ds(off, C)], s0)
        pltpu.sync_copy(dst_hbm.at[pl.ds(off, C)], d0)
        pltpu.async_copy(h_hbm.at[s0], buf0, gsem).wait()
        pltpu.sync_copy(buf0, agg_sh.at[d0], add=True)
        return 0

    lax.fori_loop(0, NCHUNK, body, 0)

    plsc.subcore_barrier()
    pltpu.sync_copy(agg_sh.at[pl.ds(zoff, RPS)],
                    agg_hbm.at[cid, pl.ds(zoff, RPS)])


_BLK = 256


def _sage_body(h_ref, a0_ref, a1_ref, d0_ref, d1_ref, ws_ref, wn_ref, b_ref,
               o_ref):
    deg = jnp.maximum(d0_ref[:, 0:1] + d1_ref[:, 0:1], 1.0)
    agg = (a0_ref[...] + a1_ref[...]) / deg
    acc = jnp.dot(h_ref[...], ws_ref[...], preferred_element_type=jnp.float32)
    acc += jnp.dot(agg, wn_ref[...], preferred_element_type=jnp.float32)
    o_ref[...] = jnp.maximum(acc + b_ref[...], 0.0)


def _sage(h, a0, a1, d0, d1, Ws, Wn, b):
    grid = (NPAD // _BLK,)
    return pl.pallas_call(
        _sage_body,
        grid=grid,
        in_specs=[
            pl.BlockSpec((_BLK, D), lambda i: (i, 0)),
            pl.BlockSpec((_BLK, D), lambda i: (i, 0)),
            pl.BlockSpec((_BLK, D), lambda i: (i, 0)),
            pl.BlockSpec((_BLK, D), lambda i: (i, 0)),
            pl.BlockSpec((_BLK, D), lambda i: (i, 0)),
            pl.BlockSpec((D, D), lambda i: (0, 0)),
            pl.BlockSpec((D, D), lambda i: (0, 0)),
            pl.BlockSpec((1, D), lambda i: (0, 0)),
        ],
        out_specs=pl.BlockSpec((_BLK, D), lambda i: (i, 0)),
        out_shape=jax.ShapeDtypeStruct((NPAD, D), jnp.float32),
    )(h, a0, a1, d0, d1, Ws, Wn, b)


def _cls_body(h_ref, gid_ref, wc1_ref, bc1_ref, wc2_ref, bc2_ref,
              o_ref, pooled_ref):
    def body(g, _):
        mask = gid_ref[...] == g
        vals = jnp.where(mask, h_ref[...], -jnp.inf)
        pooled_ref[pl.ds(g, 1), :] = jnp.max(vals, axis=0, keepdims=True)
        return 0

    lax.fori_loop(0, G, body, 0)
    pooled = pooled_ref[...]
    pooled = jnp.where(jnp.isfinite(pooled), pooled, 0.0)
    hid = jnp.dot(pooled, wc1_ref[...], preferred_element_type=jnp.float32)
    hid = jnp.maximum(hid + bc1_ref[...], 0.0)
    o_ref[...] = jnp.dot(hid, wc2_ref[...],
                         preferred_element_type=jnp.float32) + bc2_ref[...]


def _classifier(h2, gid, Wc1, bc1, Wc2p, bc2p):
    return pl.pallas_call(
        _cls_body,
        out_shape=jax.ShapeDtypeStruct((G, 128), jnp.float32),
        scratch_shapes=[pltpu.VMEM((G, D), jnp.float32)],
    )(h2, gid, Wc1, bc1, Wc2p, bc2p)


@jax.jit
def kernel(token_ids, edge_index, graph_ids, emb, W_self1, W_neigh1, b1,
           W_self2, W_neigh2, b2, Wc1, bc1, Wc2, bc2):
    tok = jnp.concatenate(
        [token_ids, jnp.zeros((NPAD - N,), jnp.int32)]).astype(jnp.int32)
    src = jnp.concatenate(
        [edge_index[0].astype(jnp.int32),
         jnp.arange(EPAD - E, dtype=jnp.int32) % N])
    dst = jnp.concatenate(
        [edge_index[1].astype(jnp.int32),
         N + jnp.arange(EPAD - E, dtype=jnp.int32) % (NPAD - N)])
    gid = jnp.concatenate(
        [graph_ids, jnp.full((NPAD - N,), G, jnp.int32)]).reshape(NPAD, 1)
    z = jnp.zeros((NPAD, D), jnp.float32)
    ones = jnp.ones((C, D), jnp.float32)

    x, deg = _embed_deg_k(tok, emb, dst, z, ones)
    agg1 = _agg_k(src, dst, x, z)
    h1 = _sage(x, agg1[0], agg1[1], deg[0], deg[1],
               W_self1, W_neigh1, b1.reshape(1, D))
    agg2 = _agg_k(src, dst, h1, z)
    h2 = _sage(h1, agg2[0], agg2[1], deg[0], deg[1],
               W_self2, W_neigh2, b2.reshape(1, D))

    Wc2p = jnp.pad(Wc2, ((0, 0), (0, 128 - NCLS)))
    bc2p = jnp.pad(bc2, (0, 128 - NCLS)).reshape(1, 128)
    logits = _classifier(h2, gid, Wc1, bc1.reshape(1, D), Wc2p, bc2p)
    return logits[:, :NCLS]


# speedup vs baseline: 2.6921x; 1.5454x over previous
import functools

import jax
import jax.numpy as jnp
from jax import lax
from jax.experimental import pallas as pl
from jax.experimental.pallas import tpu as pltpu
from jax.experimental.pallas import tpu_sc as plsc

N = 10000
E = 320000
D = 128
G = 64
NCLS = 10
NC, NS = 2, 16
NW = NC * NS
NPAD = 10240
RPS = NPAD // NS
RPT = NPAD // NW
TCH = 80
C = 128
NCHUNK = 80
EPAD = NW * NCHUNK * C
EPT = NCHUNK * C
EHALF = EPT // 2
NH = NCHUNK // 2

_mesh = plsc.VectorSubcoreMesh(core_axis_name="c", subcore_axis_name="s")


@functools.partial(
    pl.kernel,
    out_type=(
        jax.ShapeDtypeStruct((NPAD, D), jnp.float32),
        jax.ShapeDtypeStruct((NC, NPAD, D), jnp.float32),
    ),
    mesh=_mesh,
    scratch_types=[
        pltpu.VMEM((EHALF,), jnp.int32),
        pltpu.VMEM((TCH,), jnp.int32),
        pltpu.VMEM((TCH, D), jnp.float32),
        pltpu.VMEM((C, D), jnp.float32),
        pltpu.VMEM_SHARED((NPAD, D), jnp.float32),
        pltpu.SemaphoreType.DMA,
        pltpu.SemaphoreType.DMA,
    ],
)
def _embed_deg_k(tok_hbm, emb_hbm, dst_hbm, z_hbm, ones_hbm,
                 x_hbm, deg_hbm, didx_v, tidx_v, trows_v, ones_v, deg_sh,
                 sem0, sem1):
    cid = lax.axis_index("c")
    sid = lax.axis_index("s")
    wid = sid * NC + cid
    zoff = pl.multiple_of(sid * RPS, 8)
    pltpu.sync_copy(z_hbm.at[pl.ds(zoff, RPS)], deg_sh.at[pl.ds(zoff, RPS)])
    pltpu.sync_copy(ones_hbm, ones_v)
    plsc.subcore_barrier()

    K = 4

    def deg_half(p):
        ebase = pl.multiple_of(wid * EPT + p * EHALF, 8)
        pltpu.sync_copy(dst_hbm.at[pl.ds(ebase, EHALF)], didx_v)

        def edge_body(j, _):
            off = pl.multiple_of(j * C, 8)
            pltpu.async_copy(ones_v, deg_sh.at[didx_v.at[pl.ds(off, C)]],
                             sem0, add=True)

            @pl.when(j >= K)
            def _wait():
                pltpu.make_async_copy(
                    ones_v, deg_sh.at[didx_v.at[pl.ds(off, C)]],
                    sem0).wait()

            return 0

        lax.fori_loop(0, NCHUNK // 2, edge_body, 0)

        def drain_body(j, _):
            pltpu.make_async_copy(
                ones_v, deg_sh.at[didx_v.at[pl.ds(0, C)]], sem0).wait()
            return 0

        lax.fori_loop(0, K, drain_body, 0)

    deg_half(0)
    deg_half(1)

    rbase = wid * RPT

    def tok_body(i, _):
        off = pl.multiple_of(rbase + i * TCH, 8)
        pltpu.sync_copy(tok_hbm.at[pl.ds(off, TCH)], tidx_v)
        pltpu.async_copy(emb_hbm.at[tidx_v], trows_v, sem1).wait()
        pltpu.sync_copy(trows_v, x_hbm.at[pl.ds(off, TCH)])
        return 0

    lax.fori_loop(0, RPT // TCH, tok_body, 0)

    plsc.subcore_barrier()
    pltpu.sync_copy(deg_sh.at[pl.ds(zoff, RPS)],
                    deg_hbm.at[cid, pl.ds(zoff, RPS)])


@functools.partial(
    pl.kernel,
    out_type=jax.ShapeDtypeStruct((NC, NPAD, D), jnp.float32),
    mesh=_mesh,
    scratch_types=[
        pltpu.VMEM((C,), jnp.int32),
        pltpu.VMEM((C,), jnp.int32),
        pltpu.VMEM((C,), jnp.int32),
        pltpu.VMEM((C,), jnp.int32),
        pltpu.VMEM((C, D), jnp.float32),
        pltpu.VMEM((C, D), jnp.float32),
        pltpu.VMEM_SHARED((NPAD, D), jnp.float32),
        pltpu.SemaphoreType.DMA,
        pltpu.SemaphoreType.DMA,
    ],
)
def _agg_k(src_hbm, dst_hbm, h_hbm, z_hbm,
           agg_hbm, s0, s1, d0, d1, buf0, buf1, agg_sh, sem0, sem1):
    cid = lax.axis_index("c")
    sid = lax.axis_index("s")
    wid = sid * NC + cid
    zoff = pl.multiple_of(sid * RPS, 8)
    pltpu.sync_copy(z_hbm.at[pl.ds(zoff, RPS)], agg_sh.at[pl.ds(zoff, RPS)])
    plsc.subcore_barrier()

    ebase = wid * EPT
    pltpu.sync_copy(src_hbm.at[pl.ds(pl.multiple_of(ebase, 8), C)], s0)
    pltpu.sync_copy(dst_hbm.at[pl.ds(pl.multiple_of(ebase, 8), C)], d0)
    pltpu.async_copy(h_hbm.at[s0], buf0, sem0)

    def body(j, _):
        o1 = pl.multiple_of(ebase + (2 * j + 1) * C, 8)
        pltpu.sync_copy(src_hbm.at[pl.ds(o1, C)], s1)
        pltpu.sync_copy(dst_hbm.at[pl.ds(o1, C)], d1)
        pltpu.async_copy(h_hbm.at[s1], buf1, sem1)
        pltpu.make_async_copy(h_hbm.at[s0], buf0, sem0).wait()
        pltpu.sync_copy(buf0, agg_sh.at[d0], add=True)

        @pl.when(j < NH - 1)
        def _next():
            o2 = pl.multiple_of(ebase + (2 * j + 2) * C, 8)
            pltpu.sync_copy(src_hbm.at[pl.ds(o2, C)], s0)
            pltpu.sync_copy(dst_hbm.at[pl.ds(o2, C)], d0)
            pltpu.async_copy(h_hbm.at[s0], buf0, sem0)

        pltpu.make_async_copy(h_hbm.at[s1], buf1, sem1).wait()
        pltpu.sync_copy(buf1, agg_sh.at[d1], add=True)
        return 0

    lax.fori_loop(0, NH, body, 0)

    plsc.subcore_barrier()
    pltpu.sync_copy(agg_sh.at[pl.ds(zoff, RPS)],
                    agg_hbm.at[cid, pl.ds(zoff, RPS)])


_BLK = 256


def _sage_body(h_ref, a0_ref, a1_ref, d0_ref, d1_ref, ws_ref, wn_ref, b_ref,
               o_ref):
    deg = jnp.maximum(d0_ref[:, 0:1] + d1_ref[:, 0:1], 1.0)
    agg = (a0_ref[...] + a1_ref[...]) / deg
    acc = jnp.dot(h_ref[...], ws_ref[...], preferred_element_type=jnp.float32)
    acc += jnp.dot(agg, wn_ref[...], preferred_element_type=jnp.float32)
    o_ref[...] = jnp.maximum(acc + b_ref[...], 0.0)


def _sage(h, a0, a1, d0, d1, Ws, Wn, b):
    grid = (NPAD // _BLK,)
    return pl.pallas_call(
        _sage_body,
        grid=grid,
        in_specs=[
            pl.BlockSpec((_BLK, D), lambda i: (i, 0)),
            pl.BlockSpec((_BLK, D), lambda i: (i, 0)),
            pl.BlockSpec((_BLK, D), lambda i: (i, 0)),
            pl.BlockSpec((_BLK, D), lambda i: (i, 0)),
            pl.BlockSpec((_BLK, D), lambda i: (i, 0)),
            pl.BlockSpec((D, D), lambda i: (0, 0)),
            pl.BlockSpec((D, D), lambda i: (0, 0)),
            pl.BlockSpec((1, D), lambda i: (0, 0)),
        ],
        out_specs=pl.BlockSpec((_BLK, D), lambda i: (i, 0)),
        out_shape=jax.ShapeDtypeStruct((NPAD, D), jnp.float32),
    )(h, a0, a1, d0, d1, Ws, Wn, b)


def _cls_body(h_ref, gid_ref, wc1_ref, bc1_ref, wc2_ref, bc2_ref,
              o_ref, pooled_ref):
    def body(g, _):
        mask = gid_ref[...] == g
        vals = jnp.where(mask, h_ref[...], -jnp.inf)
        pooled_ref[pl.ds(g, 1), :] = jnp.max(vals, axis=0, keepdims=True)
        return 0

    lax.fori_loop(0, G, body, 0)
    pooled = pooled_ref[...]
    pooled = jnp.where(jnp.isfinite(pooled), pooled, 0.0)
    hid = jnp.dot(pooled, wc1_ref[...], preferred_element_type=jnp.float32)
    hid = jnp.maximum(hid + bc1_ref[...], 0.0)
    o_ref[...] = jnp.dot(hid, wc2_ref[...],
                         preferred_element_type=jnp.float32) + bc2_ref[...]


def _classifier(h2, gid, Wc1, bc1, Wc2p, bc2p):
    return pl.pallas_call(
        _cls_body,
        out_shape=jax.ShapeDtypeStruct((G, 128), jnp.float32),
        scratch_shapes=[pltpu.VMEM((G, D), jnp.float32)],
    )(h2, gid, Wc1, bc1, Wc2p, bc2p)


@jax.jit
def kernel(token_ids, edge_index, graph_ids, emb, W_self1, W_neigh1, b1,
           W_self2, W_neigh2, b2, Wc1, bc1, Wc2, bc2):
    tok = jnp.concatenate(
        [token_ids, jnp.zeros((NPAD - N,), jnp.int32)]).astype(jnp.int32)
    src = jnp.concatenate(
        [edge_index[0].astype(jnp.int32),
         jnp.arange(EPAD - E, dtype=jnp.int32) % N])
    dst = jnp.concatenate(
        [edge_index[1].astype(jnp.int32),
         N + jnp.arange(EPAD - E, dtype=jnp.int32) % (NPAD - N)])
    gid = jnp.concatenate(
        [graph_ids, jnp.full((NPAD - N,), G, jnp.int32)]).reshape(NPAD, 1)
    z = jnp.zeros((NPAD, D), jnp.float32)
    ones = jnp.ones((C, D), jnp.float32)

    x, deg = _embed_deg_k(tok, emb, dst, z, ones)
    agg1 = _agg_k(src, dst, x, z)
    h1 = _sage(x, agg1[0], agg1[1], deg[0], deg[1],
               W_self1, W_neigh1, b1.reshape(1, D))
    agg2 = _agg_k(src, dst, h1, z)
    h2 = _sage(h1, agg2[0], agg2[1], deg[0], deg[1],
               W_self2, W_neigh2, b2.reshape(1, D))

    Wc2p = jnp.pad(Wc2, ((0, 0), (0, 128 - NCLS)))
    bc2p = jnp.pad(bc2, (0, 128 - NCLS)).reshape(1, 128)
    logits = _classifier(h2, gid, Wc1, bc1.reshape(1, D), Wc2p, bc2p)
    return logits[:, :NCLS]
